# baseline (speedup 1.0000x reference)
import jax
import jax.numpy as jnp
import numpy as np
from jax import lax
from jax.experimental import pallas as pl
from jax.experimental.pallas import tpu as pltpu
from jax.experimental.pallas import tpu_sc as plsc

H = 384
W = 384
C = 96
B = 4
HW = H * W
NSEC = 12
OC = HW // NSEC
NSC = 2
NT = 16
BPS = B // NSC
PPT = HW // NT
CH = 128
NCH = PPT // CH
OPT = OC // NT
ZR = OPT // 8
CAP = 1152
CLAMP = CAP - CH


def _lin_idx_body(y_ref, x_ref, o_ref):
    y = (y_ref[...] * np.float32(H)).astype(jnp.int32)
    x = (x_ref[...] * np.float32(W)).astype(jnp.int32)
    o_ref[...] = y * np.int32(W) + x


def _scatter_body(img, lin, out, rows_v, rows_w, idx_v, zero_v, pid_v, tgt_v,
                  pid_w, tgt_w, lst, sem0, sem1, sem_z, sem_d, acc):
    cid = lax.axis_index("c")
    sid = lax.axis_index("s")
    pbase = sid * PPT
    abase = sid * OPT
    lane = lax.iota(jnp.int32, 16)
    zeros16 = jnp.zeros((16,), jnp.float32)
    m1_16 = jnp.full((16,), -1, jnp.int32)

    @pl.loop(0, ZR * C // 16)
    def _zero_fill(i):
        zero_v[i // (C // 16), pl.ds((i % (C // 16)) * 16, 16)] = zeros16

    @pl.loop(0, BPS)
    def _batch_loop(bb):
        b = cid * BPS + bb
        pltpu.sync_copy(lin.at[b, sid], idx_v)

        @pl.loop(0, NCH * (CH // 16), init_carry=(0,) * NSEC, unroll=2)
        def _bin_loop(i, cnt):
            v = idx_v[i // (CH // 16), pl.ds((i % (CH // 16)) * 16, 16)]
            p = pbase + i * 16 + lane
            sec = ((v >> 12) * 21846) >> 16
            new = []
            for s in range(NSEC):
                m = sec == s
                plsc.store_compressed(
                    lst.at[pl.ds(s * CAP + cnt[s], 16)], p, mask=m
                )
                n = jnp.max(plsc.all_reduce_population_count(m))
                new.append(jnp.minimum(cnt[s] + n, CLAMP))
            return tuple(new)

        cnt = _bin_loop
        for s in range(NSEC):
            for k in range(CH // 16):
                lst[pl.ds(s * CAP + cnt[s] + k * 16, 16)] = m1_16

        def _drain_copies(bv, sv):
            h0 = (sv * OC + abase) // W
            return [
                (acc.at[pl.ds(abase + k * W, W)], out.at[bv, h0 + k])
                for k in range(OPT // W)
            ]

        for s in range(NSEC):
            obase = s * OC
            if s > 0:
                for src_r, dst_r in _drain_copies(b, s - 1):
                    pltpu.make_async_copy(src_r, dst_r, sem_d).wait()
            else:
                @pl.when(bb > 0)
                def _wait_prev_batch_drain():
                    for src_r, dst_r in _drain_copies(b - 1, NSEC - 1):
                        pltpu.make_async_copy(src_r, dst_r, sem_d).wait()
            for k in range(OPT // ZR):
                pltpu.async_copy(
                    zero_v, acc.at[pl.ds(abase + k * ZR, ZR)], sem_z
                )
            for k in range(OPT // ZR):
                pltpu.make_async_copy(
                    zero_v, acc.at[pl.ds(abase + k * ZR, ZR)], sem_z
                ).wait()
            plsc.subcore_barrier()

            def _stage(j, pid_ref, tgt_ref):
                for k in range(CH // 16):
                    pid = lst[pl.ds(s * CAP + j * CH + k * 16, 16)]
                    local = jnp.maximum(pid - pbase, 0)
                    v = plsc.load_gather(
                        idx_v, [local >> 7, local & (CH - 1)]
                    )
                    pid_ref[pl.ds(k * 16, 16)] = pid
                    tgt_ref[pl.ds(k * 16, 16)] = jnp.where(
                        pid < 0, -1, v - obase
                    )

            def _gather(pid_ref, rows_ref, sem):
                return pltpu.async_copy(
                    img.at[b].at[plsc.Indices(pid_ref, ignored_value=-1)],
                    rows_ref,
                    sem,
                )

            def _gwait(pid_ref, rows_ref, sem):
                pltpu.make_async_copy(
                    img.at[b].at[plsc.Indices(pid_ref, ignored_value=-1)],
                    rows_ref,
                    sem,
                ).wait()

            def _scatter(rows_ref, tgt_ref):
                pltpu.sync_copy(
                    rows_ref,
                    acc.at[plsc.Indices(tgt_ref, ignored_value=-1)],
                    add=True,
                )

            trip = (cnt[s] + CH - 1) // CH

            @pl.when(trip > 0)
            def _prologue():
                _stage(0, pid_v, tgt_v)
                _gather(pid_v, rows_v, sem0)

            @pl.loop(0, (trip + 1) // 2)
            def _pair(t):
                j0 = 2 * t

                @pl.when(j0 + 1 < trip)
                def _():
                    _stage(j0 + 1, pid_w, tgt_w)
                    _gather(pid_w, rows_w, sem1)

                _gwait(pid_v, rows_v, sem0)
                _scatter(rows_v, tgt_v)

                @pl.when(j0 + 2 < trip)
                def _():
                    _stage(j0 + 2, pid_v, tgt_v)
                    _gather(pid_v, rows_v, sem0)

                @pl.when(j0 + 1 < trip)
                def _():
                    _gwait(pid_w, rows_w, sem1)
                    _scatter(rows_w, tgt_w)

            plsc.subcore_barrier()
            for src_r, dst_r in _drain_copies(b, s):
                pltpu.async_copy(src_r, dst_r, sem_d)

    fb = cid * BPS + BPS - 1
    fh0 = ((NSEC - 1) * OC + sid * OPT) // W
    for k in range(OPT // W):
        pltpu.make_async_copy(
            acc.at[pl.ds(sid * OPT + k * W, W)],
            out.at[fb, fh0 + k],
            sem_d,
        ).wait()


def kernel(image, index):
    b, h, w, c = image.shape
    y = index[..., 0].reshape(b, HW)
    x = index[..., 1].reshape(b, HW)
    lin = pl.pallas_call(
        _lin_idx_body,
        out_shape=jax.ShapeDtypeStruct((b, HW), jnp.int32),
    )(y, x)
    lin = lin.reshape(b, NT, NCH, CH)
    img2 = image.reshape(b, HW, c)

    sc_fn = pl.kernel(
        _scatter_body,
        out_type=jax.ShapeDtypeStruct((b, h, w, c), jnp.float32),
        mesh=plsc.VectorSubcoreMesh(core_axis_name="c", subcore_axis_name="s"),
        compiler_params=pltpu.CompilerParams(use_tc_tiling_on_sc=False, needs_layout_passes=False),
        scratch_types=[
            pltpu.VMEM((CH, C), jnp.float32),
            pltpu.VMEM((CH, C), jnp.float32),
            pltpu.VMEM((NCH, CH), jnp.int32),
            pltpu.VMEM((ZR, C), jnp.float32),
            pltpu.VMEM((CH,), jnp.int32),
            pltpu.VMEM((CH,), jnp.int32),
            pltpu.VMEM((CH,), jnp.int32),
            pltpu.VMEM((CH,), jnp.int32),
            pltpu.VMEM((NSEC * CAP,), jnp.int32),
            pltpu.SemaphoreType.DMA,
            pltpu.SemaphoreType.DMA,
            pltpu.SemaphoreType.DMA,
            pltpu.SemaphoreType.DMA,
            pltpu.VMEM_SHARED((OC, C), jnp.float32),
        ],
    )
    return sc_fn(img2, lin)


# speedup vs baseline: 1.0042x; 1.0042x over previous
import jax
import jax.numpy as jnp
import numpy as np
from jax import lax
from jax.experimental import pallas as pl
from jax.experimental.pallas import tpu as pltpu
from jax.experimental.pallas import tpu_sc as plsc

H = 384
W = 384
C = 96
B = 4
HW = H * W
NSEC = 12
OC = HW // NSEC
NSC = 2
NT = 16
BPS = B // NSC
PPT = HW // NT
CH = 128
NCH = PPT // CH
OPT = OC // NT
ZR = OPT // 8
CAP = 1152
CLAMP = CAP - CH


def _lin_idx_body(y_ref, x_ref, o_ref):
    y = (y_ref[...] * np.float32(H)).astype(jnp.int32)
    x = (x_ref[...] * np.float32(W)).astype(jnp.int32)
    o_ref[...] = y * np.int32(W) + x


def _scatter_body(img, lin, out, rows_v, rows_w, idx_v, zero_v, pid_v, tgt_v,
                  pid_w, tgt_w, lst, sem0, sem1, sem_z, sem_d, acc):
    cid = lax.axis_index("c")
    sid = lax.axis_index("s")
    pbase = sid * PPT
    abase = sid * OPT
    lane = lax.iota(jnp.int32, 16)
    zeros16 = jnp.zeros((16,), jnp.float32)
    m1_16 = jnp.full((16,), -1, jnp.int32)

    @pl.loop(0, ZR * C // 16)
    def _zero_fill(i):
        zero_v[i // (C // 16), pl.ds((i % (C // 16)) * 16, 16)] = zeros16

    @pl.loop(0, BPS)
    def _batch_loop(bb):
        b = cid * BPS + bb
        pltpu.sync_copy(lin.at[b, sid], idx_v)

        @pl.loop(0, NCH * (CH // 16), init_carry=(0,) * NSEC)
        def _bin_loop(i, cnt):
            v = idx_v[i // (CH // 16), pl.ds((i % (CH // 16)) * 16, 16)]
            p = pbase + i * 16 + lane
            sec = ((v >> 12) * 21846) >> 16
            new = []
            for s in range(NSEC):
                m = sec == s
                plsc.store_compressed(
                    lst.at[pl.ds(s * CAP + cnt[s], 16)], p, mask=m
                )
                n = jnp.max(plsc.all_reduce_population_count(m))
                new.append(jnp.minimum(cnt[s] + n, CLAMP))
            return tuple(new)

        cnt = _bin_loop
        for s in range(NSEC):
            for k in range(CH // 16):
                lst[pl.ds(s * CAP + cnt[s] + k * 16, 16)] = m1_16

        def _drain_copies(bv, sv):
            h0 = (sv * OC + abase) // W
            return [
                (acc.at[pl.ds(abase + k * W, W)], out.at[bv, h0 + k])
                for k in range(OPT // W)
            ]

        for s in range(NSEC):
            obase = s * OC
            if s > 0:
                for src_r, dst_r in _drain_copies(b, s - 1):
                    pltpu.make_async_copy(src_r, dst_r, sem_d).wait()
            else:
                @pl.when(bb > 0)
                def _wait_prev_batch_drain():
                    for src_r, dst_r in _drain_copies(b - 1, NSEC - 1):
                        pltpu.make_async_copy(src_r, dst_r, sem_d).wait()
            for k in range(OPT // ZR):
                pltpu.async_copy(
                    zero_v, acc.at[pl.ds(abase + k * ZR, ZR)], sem_z
                )
            for k in range(OPT // ZR):
                pltpu.make_async_copy(
                    zero_v, acc.at[pl.ds(abase + k * ZR, ZR)], sem_z
                ).wait()
            plsc.subcore_barrier()

            def _stage(j, pid_ref, tgt_ref):
                for k in range(CH // 16):
                    pid = lst[pl.ds(s * CAP + j * CH + k * 16, 16)]
                    local = jnp.maximum(pid - pbase, 0)
                    v = plsc.load_gather(
                        idx_v, [local >> 7, local & (CH - 1)]
                    )
                    pid_ref[pl.ds(k * 16, 16)] = pid
                    tgt_ref[pl.ds(k * 16, 16)] = jnp.where(
                        pid < 0, -1, v - obase
                    )

            def _gather(pid_ref, rows_ref, sem):
                return pltpu.async_copy(
                    img.at[b].at[plsc.Indices(pid_ref, ignored_value=-1)],
                    rows_ref,
                    sem,
                )

            def _gwait(pid_ref, rows_ref, sem):
                pltpu.make_async_copy(
                    img.at[b].at[plsc.Indices(pid_ref, ignored_value=-1)],
                    rows_ref,
                    sem,
                ).wait()

            def _scatter(rows_ref, tgt_ref):
                pltpu.sync_copy(
                    rows_ref,
                    acc.at[plsc.Indices(tgt_ref, ignored_value=-1)],
                    add=True,
                )

            trip = (cnt[s] + CH - 1) // CH

            @pl.when(trip > 0)
            def _prologue():
                _stage(0, pid_v, tgt_v)
                _gather(pid_v, rows_v, sem0)

            @pl.loop(0, (trip + 1) // 2)
            def _pair(t):
                j0 = 2 * t

                @pl.when(j0 + 1 < trip)
                def _():
                    _stage(j0 + 1, pid_w, tgt_w)
                    _gather(pid_w, rows_w, sem1)

                _gwait(pid_v, rows_v, sem0)
                _scatter(rows_v, tgt_v)

                @pl.when(j0 + 2 < trip)
                def _():
                    _stage(j0 + 2, pid_v, tgt_v)
                    _gather(pid_v, rows_v, sem0)

                @pl.when(j0 + 1 < trip)
                def _():
                    _gwait(pid_w, rows_w, sem1)
                    _scatter(rows_w, tgt_w)

            plsc.subcore_barrier()
            for src_r, dst_r in _drain_copies(b, s):
                pltpu.async_copy(src_r, dst_r, sem_d)

    fb = cid * BPS + BPS - 1
    fh0 = ((NSEC - 1) * OC + sid * OPT) // W
    for k in range(OPT // W):
        pltpu.make_async_copy(
            acc.at[pl.ds(sid * OPT + k * W, W)],
            out.at[fb, fh0 + k],
            sem_d,
        ).wait()


def kernel(image, index):
    b, h, w, c = image.shape
    y = index[..., 0].reshape(b, HW)
    x = index[..., 1].reshape(b, HW)
    lin = pl.pallas_call(
        _lin_idx_body,
        out_shape=jax.ShapeDtypeStruct((b, HW), jnp.int32),
    )(y, x)
    lin = lin.reshape(b, NT, NCH, CH)
    img2 = image.reshape(b, HW, c)

    sc_fn = pl.kernel(
        _scatter_body,
        out_type=jax.ShapeDtypeStruct((b, h, w, c), jnp.float32),
        mesh=plsc.VectorSubcoreMesh(core_axis_name="c", subcore_axis_name="s"),
        compiler_params=pltpu.CompilerParams(use_tc_tiling_on_sc=False, needs_layout_passes=False),
        scratch_types=[
            pltpu.VMEM((CH, C), jnp.float32),
            pltpu.VMEM((CH, C), jnp.float32),
            pltpu.VMEM((NCH, CH), jnp.int32),
            pltpu.VMEM((ZR, C), jnp.float32),
            pltpu.VMEM((CH,), jnp.int32),
            pltpu.VMEM((CH,), jnp.int32),
            pltpu.VMEM((CH,), jnp.int32),
            pltpu.VMEM((CH,), jnp.int32),
            pltpu.VMEM((NSEC * CAP,), jnp.int32),
            pltpu.SemaphoreType.DMA,
            pltpu.SemaphoreType.DMA,
            pltpu.SemaphoreType.DMA,
            pltpu.SemaphoreType.DMA,
            pltpu.VMEM_SHARED((OC, C), jnp.float32),
        ],
    )
    return sc_fn(img2, lin)
